# baseline (speedup 1.0000x reference)
import functools

import jax
import jax.numpy as jnp
from jax import lax
from jax.experimental import pallas as pl
from jax.experimental.pallas import tpu as pltpu
from jax.experimental.pallas import tpu_sc as plsc

_B = 16384
_F = 26
_V = 1000000
_D = 13

_NC = 2
_NS = 16
_NW = _NC * _NS
_BPW = _B // _NW
_CH = 128
_NCH = _F * _BPW // _CH
_CPF = _BPW // _CH


def _sc_embed_linear(table_flat, idx_blocks, dense_blocks, wb):
    mesh = plsc.VectorSubcoreMesh(core_axis_name="c", subcore_axis_name="s")

    @functools.partial(
        pl.kernel,
        mesh=mesh,
        out_type=jax.ShapeDtypeStruct((_B,), jnp.float32),
        scratch_types=[
            pltpu.VMEM((_NCH, _CH), jnp.int32),
            pltpu.VMEM((_NCH, _CH), jnp.float32),
            pltpu.VMEM((_D, _BPW), jnp.float32),
            pltpu.VMEM((_F + _D + 1, 16), jnp.float32),
            pltpu.VMEM((_BPW,), jnp.float32),
            pltpu.SemaphoreType.DMA,
        ],
    )
    def k(table_hbm, idx_hbm, dense_hbm, wb_hbm, out_hbm,
          idx_v, g_v, dense_v, wb_v, out_v, sem):
        wid = lax.axis_index("s") * _NC + lax.axis_index("c")

        pltpu.sync_copy(idx_hbm.at[wid], idx_v)

        def fire(j, carry):
            pltpu.make_async_copy(
                table_hbm.at[idx_v.at[j]], g_v.at[j], sem).start()
            return carry
        lax.fori_loop(0, _NCH, fire, 0)

        pltpu.sync_copy(dense_hbm.at[wid], dense_v)
        pltpu.sync_copy(wb_hbm, wb_v)

        def dense_part(s, carry):
            off = s * 16
            acc = wb_v[_F + _D]
            for d in range(_D):
                acc = acc + dense_v[d, pl.ds(off, 16)] * wb_v[_F + d]
            out_v[pl.ds(off, 16)] = acc
            return carry
        lax.fori_loop(0, _BPW // 16, dense_part, 0)

        def drain(j, carry):
            pltpu.make_async_copy(
                table_hbm.at[idx_v.at[j]], g_v.at[j], sem).wait()
            return carry
        lax.fori_loop(0, _NCH, drain, 0)

        def gather_part(f, carry):
            wv = wb_v[f]
            for c in range(_CPF):
                j = f * _CPF + c
                for s in range(_CH // 16):
                    o = c * _CH + s * 16
                    out_v[pl.ds(o, 16)] = (
                        out_v[pl.ds(o, 16)] + g_v[j, pl.ds(s * 16, 16)] * wv)
            return carry
        lax.fori_loop(0, _F, gather_part, 0)

        pltpu.sync_copy(out_v, out_hbm.at[pl.ds(wid * _BPW, _BPW)])

    return k(table_flat, idx_blocks, dense_blocks, wb)


def kernel(sparse_idx, dense, emb_tables, fc_w, fc_b):
    table_flat = emb_tables.reshape(_F * _V)
    flat_idx = sparse_idx + jnp.arange(_F, dtype=jnp.int32)[None, :] * _V
    idx_blocks = (flat_idx.T.reshape(_F, _NW, _BPW)
                  .transpose(1, 0, 2).reshape(_NW, _NCH, _CH))
    dense_blocks = dense.T.reshape(_D, _NW, _BPW).transpose(1, 0, 2)
    wb = jnp.broadcast_to(
        jnp.concatenate([fc_w.reshape(-1), fc_b]).reshape(_F + _D + 1, 1),
        (_F + _D + 1, 16))
    out = _sc_embed_linear(table_flat, idx_blocks, dense_blocks, wb)
    return out.reshape(_B, 1)
